# Initial kernel scaffold; baseline (speedup 1.0000x reference)
import jax
import jax.numpy as jnp
from jax.experimental import pallas as pl


def kernel(X, edge_weight, edge_index, batch):
    raise NotImplementedError("write your pallas kernel here")


# speedup vs baseline: 6.5890x; 6.5890x over previous
import functools

import jax
import jax.numpy as jnp
from jax import lax
from jax.experimental import pallas as pl
from jax.experimental.pallas import tpu as pltpu
from jax.experimental.pallas import tpu_sc as plsc

N = 10000
D = 128
E = 320000
G = 64

NC = 2
NS = 16
NW = NC * NS
CHUNK = 128
N_CHUNKS = -(-E // (NW * CHUNK))
EPW = N_CHUNKS * CHUNK
E_PAD = NW * EPW
N_PAD = 10240
ROWS_PER_TILE = N_PAD // NS


def _conv_body(x_hbm, zeros_hbm, src_hbm, dst_hbm, w_hbm, out_hbm,
               acc_ref, src_v, dst_v, w_v, rows_v, sem):
    cid = lax.axis_index("c")
    sid = lax.axis_index("s")
    wid = cid * NS + sid

    pltpu.sync_copy(src_hbm.at[wid], src_v)
    pltpu.sync_copy(dst_hbm.at[wid], dst_v)
    pltpu.sync_copy(w_hbm.at[wid], w_v)

    pltpu.sync_copy(zeros_hbm.at[pl.ds(sid * ROWS_PER_TILE, ROWS_PER_TILE)],
                    acc_ref.at[pl.ds(sid * ROWS_PER_TILE, ROWS_PER_TILE)])
    plsc.subcore_barrier()

    def chunk_body(j, carry):
        pltpu.async_copy(x_hbm.at[src_v.at[j]], rows_v, sem).wait()

        def scale_grp(g, c):
            wv = w_v[j, pl.ds(g * 16, 16)]
            for l in range(16):
                w = wv[l]
                i = g * 16 + l
                for kk in range(D // 16):
                    sl = pl.ds(kk * 16, 16)
                    rows_v[i, sl] = rows_v[i, sl] * w
            return c

        lax.fori_loop(0, CHUNK // 16, scale_grp, 0, unroll=False)

        pltpu.sync_copy(rows_v, acc_ref.at[dst_v.at[j]], add=True)
        return carry

    lax.fori_loop(0, N_CHUNKS, chunk_body, 0, unroll=False)
    plsc.subcore_barrier()

    pltpu.sync_copy(acc_ref.at[pl.ds(sid * ROWS_PER_TILE, ROWS_PER_TILE)],
                    out_hbm.at[cid, pl.ds(sid * ROWS_PER_TILE, ROWS_PER_TILE)])


def _make_conv():
    mesh = plsc.VectorSubcoreMesh(core_axis_name="c", subcore_axis_name="s",
                                  num_cores=NC, num_subcores=NS)

    return pl.kernel(
        _conv_body,
        out_type=jax.ShapeDtypeStruct((NC, N_PAD, D), jnp.float32),
        mesh=mesh,
        scratch_types=[
            pltpu.VMEM_SHARED((N_PAD, D), jnp.float32),
            pltpu.VMEM((N_CHUNKS, CHUNK), jnp.int32),
            pltpu.VMEM((N_CHUNKS, CHUNK), jnp.int32),
            pltpu.VMEM((N_CHUNKS, CHUNK), jnp.float32),
            pltpu.VMEM((CHUNK, D), jnp.float32),
            pltpu.SemaphoreType.DMA,
        ],
    )


_sc_conv = _make_conv()


def _add_body(p_ref, o_ref):
    o_ref[...] = p_ref[0] + p_ref[1]


def _tc_add(partials):
    blk = 1024
    return pl.pallas_call(
        _add_body,
        grid=(N_PAD // blk,),
        in_specs=[pl.BlockSpec((NC, blk, D), lambda i: (0, i, 0))],
        out_specs=pl.BlockSpec((blk, D), lambda i: (i, 0)),
        out_shape=jax.ShapeDtypeStruct((N_PAD, D), jnp.float32),
    )(partials)


_POOL_BLK = 200
_POOL_STEPS = N // _POOL_BLK


def _pool_body(b_ref, y1, y2, y3, y4, y5, y6, o_ref, macc, ccnt):
    i = pl.program_id(0)

    a1, a2, a3 = y1[...], y2[...], y3[...]
    a4, a5, a6 = y4[...], y5[...], y6[...]
    f = jnp.concatenate(
        [a4,
         jnp.abs(a1 - a2), jnp.abs(a2 - a4),
         jnp.abs(a3 - a2), jnp.abs(a5 - a3), jnp.abs(a6 - a4)], axis=1)

    seg = b_ref[0]
    gids = lax.broadcasted_iota(jnp.int32, (G, _POOL_BLK), 0)
    m = (seg == gids).astype(jnp.float32)

    part = jnp.dot(m, f, preferred_element_type=jnp.float32)
    cnt = jnp.sum(m, axis=1, keepdims=True)

    @pl.when(i == 0)
    def _init():
        macc[...] = part
        ccnt[...] = cnt

    @pl.when(i > 0)
    def _accum():
        macc[...] += part
        ccnt[...] += cnt

    @pl.when(i == _POOL_STEPS - 1)
    def _final():
        o_ref[...] = macc[...] / jnp.maximum(ccnt[...], 1.0)


def _tc_pool(batch, ys):
    b3 = batch.reshape(_POOL_STEPS, 1, _POOL_BLK)
    yspec = pl.BlockSpec((_POOL_BLK, D), lambda i: (i, 0))
    return pl.pallas_call(
        _pool_body,
        grid=(_POOL_STEPS,),
        in_specs=[pl.BlockSpec((1, 1, _POOL_BLK), lambda i: (i, 0, 0))]
        + [yspec] * 6,
        out_specs=pl.BlockSpec((G, 6 * D), lambda i: (0, 0)),
        out_shape=jax.ShapeDtypeStruct((G, 6 * D), jnp.float32),
        scratch_shapes=[
            pltpu.VMEM((G, 6 * D), jnp.float32),
            pltpu.VMEM((G, 1), jnp.float32),
        ],
    )(b3, *ys)


@jax.jit
def kernel(X, edge_weight, edge_index, batch):
    pad = E_PAD - E
    src = jnp.concatenate([edge_index[0], jnp.zeros((pad,), jnp.int32)])
    dst = jnp.concatenate([edge_index[1], jnp.zeros((pad,), jnp.int32)])
    w = jnp.concatenate([edge_weight, jnp.zeros((pad,), jnp.float32)])
    src = src.reshape(NW, N_CHUNKS, CHUNK)
    dst = dst.reshape(NW, N_CHUNKS, CHUNK)
    w = w.reshape(NW, N_CHUNKS, CHUNK)
    zeros = jnp.zeros((N_PAD, D), jnp.float32)

    ys = []
    y = jnp.concatenate([X, jnp.zeros((N_PAD - N, D), jnp.float32)])
    for _ in range(6):
        partials = _sc_conv(y, zeros, src, dst, w)
        y = _tc_add(partials)
        ys.append(y)

    return _tc_pool(batch, ys)
